# Initial kernel scaffold; baseline (speedup 1.0000x reference)
import jax
import jax.numpy as jnp
from jax.experimental import pallas as pl


def kernel(input_mixed, ref_panel):
    raise NotImplementedError("write your pallas kernel here")


# speedup vs baseline: 11.1125x; 11.1125x over previous
import functools

import jax
import jax.numpy as jnp
from jax.experimental import pallas as pl
from jax.experimental.pallas import tpu as pltpu

_K = 8
_NEG_INF = float("-inf")


def _topk_body(mixed_ref, panel_ref, vals_ref, idx_ref):
    x = panel_ref[0] * mixed_ref[0]
    iota = jax.lax.broadcasted_iota(jnp.int32, x.shape, 0)
    for k in range(_K):
        m = jnp.max(x, axis=0, keepdims=True)
        amax = jnp.argmax(x, axis=0)
        vals_ref[0, k, :] = m[0]
        if k == 0:
            idx_ref[0, 0, :] = amax.astype(jnp.int32)
        if k + 1 < _K:
            x = jnp.where(iota == amax[None, :], _NEG_INF, x)


@functools.partial(jax.jit, static_argnames=("lblk",))
def _run(input_mixed, ref_panel, lblk=1024):
    b, a, n, l = ref_panel.shape
    panel = ref_panel.reshape(b * a, n, l)
    mixed = input_mixed.reshape(b, 1, l)
    grid = (b * a, l // lblk)
    vals, idx = pl.pallas_call(
        _topk_body,
        grid=grid,
        in_specs=[
            pl.BlockSpec((1, 1, lblk), lambda i, j: (i // 4, 0, j)),
            pl.BlockSpec((1, n, lblk), lambda i, j: (i, 0, j)),
        ],
        out_specs=[
            pl.BlockSpec((1, _K, lblk), lambda i, j: (i, 0, j)),
            pl.BlockSpec((1, 1, lblk), lambda i, j: (i, 0, j)),
        ],
        out_shape=[
            jax.ShapeDtypeStruct((b * a, _K, l), jnp.float32),
            jax.ShapeDtypeStruct((b * a, 1, l), jnp.int32),
        ],
    )(mixed, panel)
    return vals.reshape(b, a, _K, l), idx.reshape(b, a, l)


def kernel(input_mixed, ref_panel):
    return _run(input_mixed, ref_panel)
